# baseline (speedup 1.0000x reference)
import functools

import jax
import jax.numpy as jnp
from jax import lax
from jax.experimental import pallas as pl
from jax.experimental.pallas import tpu as pltpu
from jax.experimental.pallas import tpu_sc as plsc

N = 10000
E = 160000
D_IN = 256
D_HID = 128
D_OUT = 64

NC = 2
NS = 16
NW = NC * NS
CHUNK = 64
NCH = (E + NW * CHUNK - 1) // (NW * CHUNK)
EPAD = NW * NCH * CHUNK
NP = 10240
RPT = NP // NS


def _sc_body(with_deg, d, table, srcs, dsts, zrows, zdeg, ones_in,
             out_acc, out_deg, src_v, dst_v, rows, ones_v, acc_sh,
             deg_sh, gsems, ssems, esems):
    c = lax.axis_index("c")
    s = lax.axis_index("s")
    wid = s * NC + c
    tail = (E - (NW - 1) * NCH * CHUNK) // CHUNK
    trip3 = jnp.where(wid == NW - 1, tail // 3, NCH // 3)
    trip2t = jnp.where(wid == NW - 1, (tail % 3) // 2, (NCH % 3) // 2)

    pltpu.sync_copy(zrows, acc_sh.at[pl.ds(s * RPT, RPT)])
    pltpu.sync_copy(srcs.at[wid], src_v)
    pltpu.sync_copy(dsts.at[wid], dst_v)
    if with_deg:
        pltpu.sync_copy(zdeg, deg_sh.at[pl.ds(s * RPT, RPT)])
        pltpu.sync_copy(ones_in, ones_v)
    plsc.subcore_barrier()

    def block(j, width, carry):
        gh = [pltpu.async_copy(table.at[src_v.at[j + k]], rows[k], gsems[k])
              for k in range(width)]
        handles = []
        for k in range(width):
            gh[k].wait()
            hs = pltpu.async_copy(rows[k], acc_sh.at[dst_v.at[j + k]],
                                  ssems[k], add=True)
            he = None
            if with_deg:
                he = pltpu.async_copy(ones_v, deg_sh.at[dst_v.at[j + k]],
                                      esems[k], add=True)
            handles.append((hs, he))
        for hs, he in handles:
            hs.wait()
            if he is not None:
                he.wait()
        return carry

    def tri(j3, carry):
        return block(3 * j3, 3, carry)

    def pair(p, carry):
        return block(3 * trip3 + 2 * p, 2, carry)

    lax.fori_loop(0, trip3, tri, 0)
    lax.fori_loop(0, trip2t, pair, 0)
    plsc.subcore_barrier()

    pltpu.sync_copy(acc_sh.at[pl.ds(s * RPT, RPT)],
                    out_acc.at[c, pl.ds(s * RPT, RPT)])
    if with_deg:
        pltpu.sync_copy(deg_sh.at[pl.ds(s * RPT, RPT)],
                        out_deg.at[c, pl.ds(s * RPT, RPT)])


def _make_sc_scatter(d, with_deg):
    mesh = plsc.VectorSubcoreMesh(core_axis_name="c", subcore_axis_name="s",
                                  num_cores=NC, num_subcores=NS)
    out_type = [jax.ShapeDtypeStruct((NC, NP, d), jnp.float32)]
    if with_deg:
        out_type.append(jax.ShapeDtypeStruct((NC, NP), jnp.float32))
    scratch = [
        pltpu.VMEM((NCH, CHUNK), jnp.int32),
        pltpu.VMEM((NCH, CHUNK), jnp.int32),
        pltpu.VMEM((CHUNK, d), jnp.float32),
        pltpu.VMEM((CHUNK, d), jnp.float32),
        pltpu.VMEM((CHUNK, d), jnp.float32),
        pltpu.VMEM((CHUNK,), jnp.float32),
        pltpu.VMEM_SHARED((NP, d), jnp.float32),
        pltpu.VMEM_SHARED((NP,), jnp.float32),
    ] + [pltpu.SemaphoreType.DMA] * 9

    if with_deg:
        def body(table, srcs, dsts, zrows, zdeg, ones_in, out_acc, out_deg,
                 src_v, dst_v, r0, r1, r2, ones_v, acc_sh, deg_sh, *sems):
            _sc_body(True, d, table, srcs, dsts, zrows, zdeg, ones_in,
                     out_acc, out_deg, src_v, dst_v, (r0, r1, r2), ones_v,
                     acc_sh, deg_sh, sems[0:3], sems[3:6], sems[6:9])
    else:
        def body(table, srcs, dsts, zrows, out_acc,
                 src_v, dst_v, r0, r1, r2, ones_v, acc_sh, deg_sh, *sems):
            _sc_body(False, d, table, srcs, dsts, zrows, None, None,
                     out_acc, None, src_v, dst_v, (r0, r1, r2), ones_v,
                     acc_sh, deg_sh, sems[0:3], sems[3:6], sems[6:9])

    return pl.kernel(body, out_type=out_type, mesh=mesh,
                     scratch_types=scratch)


BM = 1000


def _pre_body(x_ref, wl_ref, wr_ref, bl_ref, a_ref, r_ref):
    xb = x_ref[...]
    a_ref[...] = jnp.dot(xb, wl_ref[...], preferred_element_type=jnp.float32)
    r_ref[...] = (jnp.dot(xb, wr_ref[...], preferred_element_type=jnp.float32)
                  + bl_ref[...])


def _mid_body(sa_ref, sb_ref, da_ref, db_ref, r_ref, wl_ref, wr_ref, bl_ref,
              a_ref, rn_ref):
    invd = 1.0 / jnp.maximum(da_ref[...] + db_ref[...], 1.0)
    h = jnp.maximum((sa_ref[0] + sb_ref[0]) * invd + r_ref[...], 0.0)
    a_ref[...] = jnp.dot(h, wl_ref[...], preferred_element_type=jnp.float32)
    rn_ref[...] = (jnp.dot(h, wr_ref[...], preferred_element_type=jnp.float32)
                   + bl_ref[...])


def _mid2_body(sa_ref, sb_ref, da_ref, db_ref, r_ref, wr_ref, bl_ref,
               h_ref, rn_ref):
    invd = 1.0 / jnp.maximum(da_ref[...] + db_ref[...], 1.0)
    h = jnp.maximum((sa_ref[0] + sb_ref[0]) * invd + r_ref[...], 0.0)
    h_ref[...] = h
    rn_ref[...] = (jnp.dot(h, wr_ref[...], preferred_element_type=jnp.float32)
                   + bl_ref[...])


def _fin_body(sa_ref, sb_ref, da_ref, db_ref, r_ref, wl_ref, o_ref):
    invd = 1.0 / jnp.maximum(da_ref[...] + db_ref[...], 1.0)
    mean = (sa_ref[0] + sb_ref[0]) * invd
    o_ref[...] = (jnp.dot(mean, wl_ref[...], preferred_element_type=jnp.float32)
                  + r_ref[...])


def _row_spec(dcol):
    return pl.BlockSpec((BM, dcol), lambda i: (i, 0))


def _part_spec(dcol, core):
    return pl.BlockSpec((1, BM, dcol), lambda i, c=core: (c, i, 0))


def _full_spec(r, c):
    return pl.BlockSpec((r, c), lambda i: (0, 0))


def _tc_pre(x, wlT, wrT, bl, dout):
    din = x.shape[1]
    return pl.pallas_call(
        _pre_body,
        grid=(N // BM,),
        in_specs=[_row_spec(din), _full_spec(din, dout), _full_spec(din, dout),
                  _full_spec(1, dout)],
        out_specs=[_row_spec(dout), _row_spec(dout)],
        out_shape=[jax.ShapeDtypeStruct((N, dout), jnp.float32)] * 2,
    )(x, wlT, wrT, bl)


def _tc_mid(s2c, da, db, r, wlT, wrT, bl, dout):
    din = s2c.shape[2]
    sa = sb = s2c
    return pl.pallas_call(
        _mid_body,
        grid=(N // BM,),
        in_specs=[_part_spec(din, 0), _part_spec(din, 1), _row_spec(1),
                  _row_spec(1), _row_spec(din), _full_spec(din, dout),
                  _full_spec(din, dout), _full_spec(1, dout)],
        out_specs=[_row_spec(dout), _row_spec(dout)],
        out_shape=[jax.ShapeDtypeStruct((N, dout), jnp.float32)] * 2,
    )(sa, sb, da, db, r, wlT, wrT, bl)


def _tc_mid2(s2c, da, db, r, wrT, bl, dout):
    din = s2c.shape[2]
    sa = sb = s2c
    return pl.pallas_call(
        _mid2_body,
        grid=(N // BM,),
        in_specs=[_part_spec(din, 0), _part_spec(din, 1), _row_spec(1),
                  _row_spec(1), _row_spec(din), _full_spec(din, dout),
                  _full_spec(1, dout)],
        out_specs=[_row_spec(din), _row_spec(dout)],
        out_shape=[jax.ShapeDtypeStruct((N, din), jnp.float32),
                   jax.ShapeDtypeStruct((N, dout), jnp.float32)],
    )(sa, sb, da, db, r, wrT, bl)


def _tc_fin(s2c, da, db, r, wlT, dout):
    din = s2c.shape[2]
    sa = sb = s2c
    return pl.pallas_call(
        _fin_body,
        grid=(N // BM,),
        in_specs=[_part_spec(din, 0), _part_spec(din, 1), _row_spec(1),
                  _row_spec(1), _row_spec(dout), _full_spec(din, dout)],
        out_specs=_row_spec(dout),
        out_shape=jax.ShapeDtypeStruct((N, dout), jnp.float32),
    )(sa, sb, da, db, r, wlT)


@jax.jit
def kernel(x, edge_index, Wl0, bl0, Wr0, Wl1, bl1, Wr1, Wl2, bl2, Wr2):
    src = edge_index[0]
    dst = edge_index[1]
    pad = EPAD - E
    srcs = jnp.concatenate([src, jnp.zeros((pad,), jnp.int32)])
    dsts = jnp.concatenate([dst, jnp.full((pad,), NP - 1, jnp.int32)])
    srcs = srcs.reshape(NW, NCH, CHUNK)
    dsts = dsts.reshape(NW, NCH, CHUNK)

    zrows128 = jnp.zeros((RPT, D_HID), jnp.float32)
    zdeg = jnp.zeros((RPT,), jnp.float32)
    ones_in = jnp.ones((CHUNK,), jnp.float32)

    sc0 = _make_sc_scatter(D_HID, True)
    sc1 = _make_sc_scatter(D_HID, False)

    a0, r0 = _tc_pre(x, Wl0.T, Wr0.T, bl0[None, :], D_HID)
    s0, deg = sc0(a0, srcs, dsts, zrows128, zdeg, ones_in)
    da = deg[0, :N, None]
    db = deg[1, :N, None]

    a1, r1 = _tc_mid(s0, da, db, r0, Wl1.T, Wr1.T, bl1[None, :], D_HID)
    (s1,) = sc1(a1, srcs, dsts, zrows128)

    h2, r2 = _tc_mid2(s1, da, db, r1, Wr2.T, bl2[None, :], D_OUT)
    (s2,) = sc1(h2, srcs, dsts, zrows128)

    return _tc_fin(s2, da, db, r2, Wl2.T, D_OUT)


# speedup vs baseline: 1.0295x; 1.0295x over previous
import functools

import jax
import jax.numpy as jnp
from jax import lax
from jax.experimental import pallas as pl
from jax.experimental.pallas import tpu as pltpu
from jax.experimental.pallas import tpu_sc as plsc

N = 10000
E = 160000
D_IN = 256
D_HID = 128
D_OUT = 64

NC = 2
NS = 16
NW = NC * NS
CHUNK = 128
NCH = (E + NW * CHUNK - 1) // (NW * CHUNK)
EPAD = NW * NCH * CHUNK
NP = 10240
RPT = NP // NS


def _sc_body(with_deg, d, table, srcs, dsts, zrows, zdeg, ones_in,
             out_acc, out_deg, src_v, dst_v, rows, ones_v, acc_sh,
             deg_sh, gsems, ssems, esems):
    c = lax.axis_index("c")
    s = lax.axis_index("s")
    wid = s * NC + c
    tail = (E - (NW - 1) * NCH * CHUNK) // CHUNK
    trip2 = jnp.where(wid == NW - 1, tail // 2, NCH // 2)

    pltpu.sync_copy(zrows, acc_sh.at[pl.ds(s * RPT, RPT)])
    pltpu.sync_copy(srcs.at[wid], src_v)
    pltpu.sync_copy(dsts.at[wid], dst_v)
    if with_deg:
        pltpu.sync_copy(zdeg, deg_sh.at[pl.ds(s * RPT, RPT)])
        pltpu.sync_copy(ones_in, ones_v)
    plsc.subcore_barrier()

    def block(j, width, carry):
        gh = [pltpu.async_copy(table.at[src_v.at[j + k]], rows[k], gsems[k])
              for k in range(width)]
        handles = []
        for k in range(width):
            gh[k].wait()
            hs = pltpu.async_copy(rows[k], acc_sh.at[dst_v.at[j + k]],
                                  ssems[k], add=True)
            he = None
            if with_deg:
                he = pltpu.async_copy(ones_v, deg_sh.at[dst_v.at[j + k]],
                                      esems[k], add=True)
            handles.append((hs, he))
        for hs, he in handles:
            hs.wait()
            if he is not None:
                he.wait()
        return carry

    lax.fori_loop(0, trip2, lambda j2, cy: block(2 * j2, 2, cy), 0)
    plsc.subcore_barrier()

    pltpu.sync_copy(acc_sh.at[pl.ds(s * RPT, RPT)],
                    out_acc.at[c, pl.ds(s * RPT, RPT)])
    if with_deg:
        pltpu.sync_copy(deg_sh.at[pl.ds(s * RPT, RPT)],
                        out_deg.at[c, pl.ds(s * RPT, RPT)])


def _make_sc_scatter(d, with_deg):
    mesh = plsc.VectorSubcoreMesh(core_axis_name="c", subcore_axis_name="s",
                                  num_cores=NC, num_subcores=NS)
    out_type = [jax.ShapeDtypeStruct((NC, NP, d), jnp.float32)]
    if with_deg:
        out_type.append(jax.ShapeDtypeStruct((NC, NP), jnp.float32))
    scratch = [
        pltpu.VMEM((NCH, CHUNK), jnp.int32),
        pltpu.VMEM((NCH, CHUNK), jnp.int32),
        pltpu.VMEM((CHUNK, d), jnp.float32),
        pltpu.VMEM((CHUNK, d), jnp.float32),
        pltpu.VMEM((CHUNK,), jnp.float32),
        pltpu.VMEM_SHARED((NP, d), jnp.float32),
        pltpu.VMEM_SHARED((NP,), jnp.float32),
    ] + [pltpu.SemaphoreType.DMA] * 6

    if with_deg:
        def body(table, srcs, dsts, zrows, zdeg, ones_in, out_acc, out_deg,
                 src_v, dst_v, r0, r1, ones_v, acc_sh, deg_sh, *sems):
            _sc_body(True, d, table, srcs, dsts, zrows, zdeg, ones_in,
                     out_acc, out_deg, src_v, dst_v, (r0, r1), ones_v,
                     acc_sh, deg_sh, sems[0:2], sems[2:4], sems[4:6])
    else:
        def body(table, srcs, dsts, zrows, out_acc,
                 src_v, dst_v, r0, r1, ones_v, acc_sh, deg_sh, *sems):
            _sc_body(False, d, table, srcs, dsts, zrows, None, None,
                     out_acc, None, src_v, dst_v, (r0, r1), ones_v,
                     acc_sh, deg_sh, sems[0:2], sems[2:4], sems[4:6])

    return pl.kernel(body, out_type=out_type, mesh=mesh,
                     scratch_types=scratch)


BM = 2000


def _dotT(x, w):
    return lax.dot_general(x, w, (((1,), (1,)), ((), ())),
                           preferred_element_type=jnp.float32)


def _mm_body(x_ref, w_ref, o_ref):
    o_ref[...] = _dotT(x_ref[...], w_ref[...])


def _mmb_body(x_ref, w_ref, b_ref, o_ref):
    o_ref[...] = _dotT(x_ref[...], w_ref[...]) + b_ref[...]


def _h_body(sa_ref, sb_ref, da_ref, db_ref, r_ref, h_ref):
    invd = 1.0 / jnp.maximum(da_ref[...] + db_ref[...], 1.0)
    h_ref[...] = jnp.maximum((sa_ref[0] + sb_ref[0]) * invd + r_ref[...], 0.0)


def _ha_body(sa_ref, sb_ref, da_ref, db_ref, r_ref, w_ref, h_ref, a_ref):
    invd = 1.0 / jnp.maximum(da_ref[...] + db_ref[...], 1.0)
    h = jnp.maximum((sa_ref[0] + sb_ref[0]) * invd + r_ref[...], 0.0)
    h_ref[...] = h
    a_ref[...] = _dotT(h, w_ref[...])


def _fin_body(sa_ref, sb_ref, da_ref, db_ref, r_ref, w_ref, o_ref):
    invd = 1.0 / jnp.maximum(da_ref[...] + db_ref[...], 1.0)
    mean = (sa_ref[0] + sb_ref[0]) * invd
    o_ref[...] = _dotT(mean, w_ref[...]) + r_ref[...]


def _row_spec(dcol):
    return pl.BlockSpec((BM, dcol), lambda i: (i, 0))


def _part_spec(dcol, core):
    return pl.BlockSpec((1, BM, dcol), lambda i, c=core: (c, i, 0))


def _full_spec(r, c):
    return pl.BlockSpec((r, c), lambda i: (0, 0))


def _shape(dcol):
    return jax.ShapeDtypeStruct((N, dcol), jnp.float32)


def _tc_mm(x, w):
    dout, din = w.shape
    return pl.pallas_call(
        _mm_body, grid=(N // BM,),
        in_specs=[_row_spec(din), _full_spec(dout, din)],
        out_specs=_row_spec(dout), out_shape=_shape(dout))(x, w)


def _tc_mmb(x, w, b):
    dout, din = w.shape
    return pl.pallas_call(
        _mmb_body, grid=(N // BM,),
        in_specs=[_row_spec(din), _full_spec(dout, din), _full_spec(1, dout)],
        out_specs=_row_spec(dout), out_shape=_shape(dout))(x, w, b[None, :])


def _tc_h(s2c, da, db, r):
    din = s2c.shape[2]
    return pl.pallas_call(
        _h_body, grid=(N // BM,),
        in_specs=[_part_spec(din, 0), _part_spec(din, 1), _row_spec(1),
                  _row_spec(1), _row_spec(din)],
        out_specs=_row_spec(din), out_shape=_shape(din))(s2c, s2c, da, db, r)


def _tc_ha(s2c, da, db, r, w):
    din = s2c.shape[2]
    dout = w.shape[0]
    return pl.pallas_call(
        _ha_body, grid=(N // BM,),
        in_specs=[_part_spec(din, 0), _part_spec(din, 1), _row_spec(1),
                  _row_spec(1), _row_spec(din), _full_spec(dout, din)],
        out_specs=[_row_spec(din), _row_spec(dout)],
        out_shape=[_shape(din), _shape(dout)])(s2c, s2c, da, db, r, w)


def _tc_fin(s2c, da, db, r, w):
    din = s2c.shape[2]
    dout = w.shape[0]
    return pl.pallas_call(
        _fin_body, grid=(N // BM,),
        in_specs=[_part_spec(din, 0), _part_spec(din, 1), _row_spec(1),
                  _row_spec(1), _row_spec(dout), _full_spec(dout, din)],
        out_specs=_row_spec(dout), out_shape=_shape(dout))(
            s2c, s2c, da, db, r, w)


@jax.jit
def kernel(x, edge_index, Wl0, bl0, Wr0, Wl1, bl1, Wr1, Wl2, bl2, Wr2):
    src = edge_index[0]
    dst = edge_index[1]
    pad = EPAD - E
    srcs = jnp.concatenate([src, jnp.zeros((pad,), jnp.int32)])
    dsts = jnp.concatenate([dst, jnp.zeros((pad,), jnp.int32)])
    srcs = srcs.reshape(NW, NCH, CHUNK)
    dsts = dsts.reshape(NW, NCH, CHUNK)

    zrows128 = jnp.zeros((RPT, D_HID), jnp.float32)
    zdeg = jnp.zeros((RPT,), jnp.float32)
    ones_in = jnp.ones((CHUNK,), jnp.float32)

    sc0 = _make_sc_scatter(D_HID, True)
    sc1 = _make_sc_scatter(D_HID, False)

    a0 = _tc_mm(x, Wl0)
    s0, deg = sc0(a0, srcs, dsts, zrows128, zdeg, ones_in)
    r0 = _tc_mmb(x, Wr0, bl0)
    da = deg[0, :N, None]
    db = deg[1, :N, None]

    h1, a1 = _tc_ha(s0, da, db, r0, Wl1)
    (s1,) = sc1(a1, srcs, dsts, zrows128)
    r1 = _tc_mmb(h1, Wr1, bl1)

    h2 = _tc_h(s1, da, db, r1)
    (s2,) = sc1(h2, srcs, dsts, zrows128)
    r2 = _tc_mmb(h2, Wr2, bl2)

    return _tc_fin(s2, da, db, r2, Wl2)


# speedup vs baseline: 1.0581x; 1.0278x over previous
import functools

import jax
import jax.numpy as jnp
from jax import lax
from jax.experimental import pallas as pl
from jax.experimental.pallas import tpu as pltpu
from jax.experimental.pallas import tpu_sc as plsc

N = 10000
E = 160000
D_IN = 256
D_HID = 128
D_OUT = 64

NC = 2
NS = 16
NW = NC * NS
CHUNK = 128
NCH = (E + NW * CHUNK - 1) // (NW * CHUNK)
EPAD = NW * NCH * CHUNK
NP = 10240
RPT = NP // NS


def _sc_body(with_deg, d, table, edges, zrows, zdeg, ones_in,
             out_acc, out_deg, src_v, dst_v, rows, ones_v, acc_sh,
             deg_sh, gsems, ssems, esems):
    c = lax.axis_index("c")
    s = lax.axis_index("s")
    wid = s * NC + c
    tail = (E - (NW - 1) * NCH * CHUNK) // CHUNK
    trip2 = jnp.where(wid == NW - 1, tail // 2, NCH // 2)

    pltpu.sync_copy(zrows, rows[0])
    pltpu.sync_copy(edges.at[0, wid], src_v)
    pltpu.sync_copy(edges.at[1, wid], dst_v)
    for z in range(RPT // CHUNK):
        pltpu.sync_copy(rows[0], acc_sh.at[pl.ds(s * RPT + z * CHUNK, CHUNK)])
    if with_deg:
        pltpu.sync_copy(zdeg, deg_sh.at[pl.ds(s * RPT, RPT)])
        pltpu.sync_copy(ones_in, ones_v)
    plsc.subcore_barrier()

    def block(j, width, carry):
        gh = [pltpu.async_copy(table.at[src_v.at[j + k]], rows[k], gsems[k])
              for k in range(width)]
        handles = []
        for k in range(width):
            gh[k].wait()
            hs = pltpu.async_copy(rows[k], acc_sh.at[dst_v.at[j + k]],
                                  ssems[k], add=True)
            he = None
            if with_deg:
                he = pltpu.async_copy(ones_v, deg_sh.at[dst_v.at[j + k]],
                                      esems[k], add=True)
            handles.append((hs, he))
        for hs, he in handles:
            hs.wait()
            if he is not None:
                he.wait()
        return carry

    lax.fori_loop(0, trip2, lambda j2, cy: block(2 * j2, 2, cy), 0)
    plsc.subcore_barrier()

    pltpu.sync_copy(acc_sh.at[pl.ds(s * RPT, RPT)],
                    out_acc.at[c, pl.ds(s * RPT, RPT)])
    if with_deg:
        pltpu.sync_copy(deg_sh.at[pl.ds(s * RPT, RPT)],
                        out_deg.at[c, pl.ds(s * RPT, RPT)])


def _make_sc_scatter(d, with_deg):
    mesh = plsc.VectorSubcoreMesh(core_axis_name="c", subcore_axis_name="s",
                                  num_cores=NC, num_subcores=NS)
    out_type = [jax.ShapeDtypeStruct((NC, NP, d), jnp.float32)]
    if with_deg:
        out_type.append(jax.ShapeDtypeStruct((NC, NP), jnp.float32))
    scratch = [
        pltpu.VMEM((NCH, CHUNK), jnp.int32),
        pltpu.VMEM((NCH, CHUNK), jnp.int32),
        pltpu.VMEM((CHUNK, d), jnp.float32),
        pltpu.VMEM((CHUNK, d), jnp.float32),
        pltpu.VMEM((CHUNK,), jnp.float32),
        pltpu.VMEM_SHARED((NP, d), jnp.float32),
        pltpu.VMEM_SHARED((NP,), jnp.float32),
    ] + [pltpu.SemaphoreType.DMA] * 6

    if with_deg:
        def body(table, edges, zrows, zdeg, ones_in, out_acc, out_deg,
                 src_v, dst_v, r0, r1, ones_v, acc_sh, deg_sh, *sems):
            _sc_body(True, d, table, edges, zrows, zdeg, ones_in,
                     out_acc, out_deg, src_v, dst_v, (r0, r1), ones_v,
                     acc_sh, deg_sh, sems[0:2], sems[2:4], sems[4:6])
    else:
        def body(table, edges, zrows, out_acc,
                 src_v, dst_v, r0, r1, ones_v, acc_sh, deg_sh, *sems):
            _sc_body(False, d, table, edges, zrows, None, None,
                     out_acc, None, src_v, dst_v, (r0, r1), ones_v,
                     acc_sh, deg_sh, sems[0:2], sems[2:4], sems[4:6])

    return pl.kernel(body, out_type=out_type, mesh=mesh,
                     scratch_types=scratch)


BM = 2000


def _dotT(x, w):
    return lax.dot_general(x, w, (((1,), (1,)), ((), ())),
                           preferred_element_type=jnp.float32)


def _mm_body(x_ref, w_ref, o_ref):
    o_ref[...] = _dotT(x_ref[...], w_ref[...])


def _mmb_body(x_ref, w_ref, b_ref, o_ref):
    o_ref[...] = _dotT(x_ref[...], w_ref[...]) + b_ref[...]


def _h_body(sa_ref, sb_ref, da_ref, db_ref, r_ref, h_ref):
    invd = 1.0 / jnp.maximum(da_ref[...] + db_ref[...], 1.0)
    h_ref[...] = jnp.maximum((sa_ref[0] + sb_ref[0]) * invd + r_ref[...], 0.0)


def _ha_body(sa_ref, sb_ref, da_ref, db_ref, r_ref, w_ref, h_ref, a_ref):
    invd = 1.0 / jnp.maximum(da_ref[...] + db_ref[...], 1.0)
    h = jnp.maximum((sa_ref[0] + sb_ref[0]) * invd + r_ref[...], 0.0)
    h_ref[...] = h
    a_ref[...] = _dotT(h, w_ref[...])


def _fin_body(sa_ref, sb_ref, da_ref, db_ref, r_ref, w_ref, o_ref):
    invd = 1.0 / jnp.maximum(da_ref[...] + db_ref[...], 1.0)
    mean = (sa_ref[0] + sb_ref[0]) * invd
    o_ref[...] = _dotT(mean, w_ref[...]) + r_ref[...]


def _row_spec(dcol):
    return pl.BlockSpec((BM, dcol), lambda i: (i, 0))


def _part_spec(dcol, core):
    return pl.BlockSpec((1, BM, dcol), lambda i, c=core: (c, i, 0))


def _full_spec(r, c):
    return pl.BlockSpec((r, c), lambda i: (0, 0))


def _shape(dcol):
    return jax.ShapeDtypeStruct((N, dcol), jnp.float32)


def _tc_mm(x, w):
    dout, din = w.shape
    return pl.pallas_call(
        _mm_body, grid=(N // BM,),
        in_specs=[_row_spec(din), _full_spec(dout, din)],
        out_specs=_row_spec(dout), out_shape=_shape(dout))(x, w)


def _tc_mmb(x, w, b):
    dout, din = w.shape
    return pl.pallas_call(
        _mmb_body, grid=(N // BM,),
        in_specs=[_row_spec(din), _full_spec(dout, din), _full_spec(1, dout)],
        out_specs=_row_spec(dout), out_shape=_shape(dout))(x, w, b[None, :])


def _tc_h(s2c, da, db, r):
    din = s2c.shape[2]
    return pl.pallas_call(
        _h_body, grid=(N // BM,),
        in_specs=[_part_spec(din, 0), _part_spec(din, 1), _row_spec(1),
                  _row_spec(1), _row_spec(din)],
        out_specs=_row_spec(din), out_shape=_shape(din))(s2c, s2c, da, db, r)


def _tc_ha(s2c, da, db, r, w):
    din = s2c.shape[2]
    dout = w.shape[0]
    return pl.pallas_call(
        _ha_body, grid=(N // BM,),
        in_specs=[_part_spec(din, 0), _part_spec(din, 1), _row_spec(1),
                  _row_spec(1), _row_spec(din), _full_spec(dout, din)],
        out_specs=[_row_spec(din), _row_spec(dout)],
        out_shape=[_shape(din), _shape(dout)])(s2c, s2c, da, db, r, w)


def _tc_fin(s2c, da, db, r, w):
    din = s2c.shape[2]
    dout = w.shape[0]
    return pl.pallas_call(
        _fin_body, grid=(N // BM,),
        in_specs=[_part_spec(din, 0), _part_spec(din, 1), _row_spec(1),
                  _row_spec(1), _row_spec(dout), _full_spec(dout, din)],
        out_specs=_row_spec(dout), out_shape=_shape(dout))(
            s2c, s2c, da, db, r, w)


@jax.jit
def kernel(x, edge_index, Wl0, bl0, Wr0, Wl1, bl1, Wr1, Wl2, bl2, Wr2):
    edges = jnp.pad(edge_index, ((0, 0), (0, EPAD - E)))
    edges = edges.reshape(2, NW, NCH, CHUNK)

    zrows128 = jnp.zeros((CHUNK, D_HID), jnp.float32)
    zdeg = jnp.zeros((RPT,), jnp.float32)
    ones_in = jnp.ones((CHUNK,), jnp.float32)

    sc0 = _make_sc_scatter(D_HID, True)
    sc1 = _make_sc_scatter(D_HID, False)

    a0 = _tc_mm(x, Wl0)
    s0, deg = sc0(a0, edges, zrows128, zdeg, ones_in)
    r0 = _tc_mmb(x, Wr0, bl0)
    da = deg[0, :N, None]
    db = deg[1, :N, None]

    h1, a1 = _tc_ha(s0, da, db, r0, Wl1)
    (s1,) = sc1(a1, edges, zrows128)
    r1 = _tc_mmb(h1, Wr1, bl1)

    h2 = _tc_h(s1, da, db, r1)
    (s2,) = sc1(h2, edges, zrows128)
    r2 = _tc_mmb(h2, Wr2, bl2)

    return _tc_fin(s2, da, db, r2, Wl2)


# speedup vs baseline: 1.1001x; 1.0397x over previous
import functools

import jax
import jax.numpy as jnp
from jax import lax
from jax.experimental import pallas as pl
from jax.experimental.pallas import tpu as pltpu
from jax.experimental.pallas import tpu_sc as plsc

N = 10000
E = 160000
D_IN = 256
D_HID = 128
D_OUT = 64

NC = 2
NS = 16
NW = NC * NS
CHUNK = 128
NCH = (E + NW * CHUNK - 1) // (NW * CHUNK)
EPAD = NW * NCH * CHUNK
NP = 10240
RPT = NP // NS


def _sc_body(with_deg, d, table, edges, zrows, zdeg, ones_in,
             out_acc, out_deg, src_v, dst_v, rows, ones_v, acc_sh,
             deg_sh, gsems, ssems, esems):
    c = lax.axis_index("c")
    s = lax.axis_index("s")
    wid = s * NC + c
    tail = (E - (NW - 1) * NCH * CHUNK) // CHUNK
    trip2 = jnp.where(wid == NW - 1, tail // 2, NCH // 2)

    pltpu.sync_copy(zrows, rows[0])
    pltpu.sync_copy(edges.at[0, wid], src_v)
    pltpu.sync_copy(edges.at[1, wid], dst_v)
    for z in range(RPT // CHUNK):
        pltpu.sync_copy(rows[0], acc_sh.at[pl.ds(s * RPT + z * CHUNK, CHUNK)])
    if with_deg:
        pltpu.sync_copy(zdeg, deg_sh.at[pl.ds(s * RPT, RPT)])
        pltpu.sync_copy(ones_in, ones_v)
    plsc.subcore_barrier()

    SB = CHUNK * d * 4
    DB = CHUNK * 4

    def block(j, width, carry):
        gh = []
        for k in range(width):
            @pl.when(j >= 2)
            def _():
                pltpu.make_async_copy(table.at[src_v.at[j + k]], rows[k],
                                      ssems[k]).wait()
                if with_deg:
                    pltpu.make_async_copy(ones_in, ones_v, esems[k]).wait()
            gh.append(pltpu.async_copy(table.at[src_v.at[j + k]], rows[k],
                                       gsems[k]))
        for k in range(width):
            gh[k].wait()
            pltpu.async_copy(rows[k], acc_sh.at[dst_v.at[j + k]],
                             ssems[k], add=True)
            if with_deg:
                pltpu.async_copy(ones_v, deg_sh.at[dst_v.at[j + k]],
                                 esems[k], add=True)
        return carry

    lax.fori_loop(0, trip2, lambda j2, cy: block(2 * j2, 2, cy), 0)
    for k in range(2):
        pltpu.make_async_copy(table.at[src_v.at[0]], rows[k],
                              ssems[k]).wait()
        if with_deg:
            pltpu.make_async_copy(ones_in, ones_v, esems[k]).wait()
    plsc.subcore_barrier()

    pltpu.sync_copy(acc_sh.at[pl.ds(s * RPT, RPT)],
                    out_acc.at[c, pl.ds(s * RPT, RPT)])
    if with_deg:
        pltpu.sync_copy(deg_sh.at[pl.ds(s * RPT, RPT)],
                        out_deg.at[c, pl.ds(s * RPT, RPT)])


def _make_sc_scatter(d, with_deg):
    mesh = plsc.VectorSubcoreMesh(core_axis_name="c", subcore_axis_name="s",
                                  num_cores=NC, num_subcores=NS)
    out_type = [jax.ShapeDtypeStruct((NC, NP, d), jnp.float32)]
    if with_deg:
        out_type.append(jax.ShapeDtypeStruct((NC, NP), jnp.float32))
    scratch = [
        pltpu.VMEM((NCH, CHUNK), jnp.int32),
        pltpu.VMEM((NCH, CHUNK), jnp.int32),
        pltpu.VMEM((CHUNK, d), jnp.float32),
        pltpu.VMEM((CHUNK, d), jnp.float32),
        pltpu.VMEM((CHUNK,), jnp.float32),
        pltpu.VMEM_SHARED((NP, d), jnp.float32),
        pltpu.VMEM_SHARED((NP,), jnp.float32),
    ] + [pltpu.SemaphoreType.DMA] * 6

    if with_deg:
        def body(table, edges, zrows, zdeg, ones_in, out_acc, out_deg,
                 src_v, dst_v, r0, r1, ones_v, acc_sh, deg_sh, *sems):
            _sc_body(True, d, table, edges, zrows, zdeg, ones_in,
                     out_acc, out_deg, src_v, dst_v, (r0, r1), ones_v,
                     acc_sh, deg_sh, sems[0:2], sems[2:4], sems[4:6])
    else:
        def body(table, edges, zrows, out_acc,
                 src_v, dst_v, r0, r1, ones_v, acc_sh, deg_sh, *sems):
            _sc_body(False, d, table, edges, zrows, None, None,
                     out_acc, None, src_v, dst_v, (r0, r1), ones_v,
                     acc_sh, deg_sh, sems[0:2], sems[2:4], sems[4:6])

    return pl.kernel(body, out_type=out_type, mesh=mesh,
                     scratch_types=scratch)


BM = 2000


def _dotT(x, w):
    return lax.dot_general(x, w, (((1,), (1,)), ((), ())),
                           preferred_element_type=jnp.float32)


def _mm_body(x_ref, w_ref, o_ref):
    o_ref[...] = _dotT(x_ref[...], w_ref[...])


def _mmb_body(x_ref, w_ref, b_ref, o_ref):
    o_ref[...] = _dotT(x_ref[...], w_ref[...]) + b_ref[...]


def _h_body(sa_ref, sb_ref, da_ref, db_ref, r_ref, h_ref):
    invd = 1.0 / jnp.maximum(da_ref[...] + db_ref[...], 1.0)
    h_ref[...] = jnp.maximum((sa_ref[0] + sb_ref[0]) * invd + r_ref[...], 0.0)


def _ha_body(sa_ref, sb_ref, da_ref, db_ref, r_ref, w_ref, h_ref, a_ref):
    invd = 1.0 / jnp.maximum(da_ref[...] + db_ref[...], 1.0)
    h = jnp.maximum((sa_ref[0] + sb_ref[0]) * invd + r_ref[...], 0.0)
    h_ref[...] = h
    a_ref[...] = _dotT(h, w_ref[...])


def _fin_body(sa_ref, sb_ref, da_ref, db_ref, r_ref, w_ref, o_ref):
    invd = 1.0 / jnp.maximum(da_ref[...] + db_ref[...], 1.0)
    mean = (sa_ref[0] + sb_ref[0]) * invd
    o_ref[...] = _dotT(mean, w_ref[...]) + r_ref[...]


def _row_spec(dcol):
    return pl.BlockSpec((BM, dcol), lambda i: (i, 0))


def _part_spec(dcol, core):
    return pl.BlockSpec((1, BM, dcol), lambda i, c=core: (c, i, 0))


def _full_spec(r, c):
    return pl.BlockSpec((r, c), lambda i: (0, 0))


def _shape(dcol):
    return jax.ShapeDtypeStruct((N, dcol), jnp.float32)


def _tc_mm(x, w):
    dout, din = w.shape
    return pl.pallas_call(
        _mm_body, grid=(N // BM,),
        in_specs=[_row_spec(din), _full_spec(dout, din)],
        out_specs=_row_spec(dout), out_shape=_shape(dout))(x, w)


def _tc_mmb(x, w, b):
    dout, din = w.shape
    return pl.pallas_call(
        _mmb_body, grid=(N // BM,),
        in_specs=[_row_spec(din), _full_spec(dout, din), _full_spec(1, dout)],
        out_specs=_row_spec(dout), out_shape=_shape(dout))(x, w, b[None, :])


def _tc_h(s2c, da, db, r):
    din = s2c.shape[2]
    return pl.pallas_call(
        _h_body, grid=(N // BM,),
        in_specs=[_part_spec(din, 0), _part_spec(din, 1), _row_spec(1),
                  _row_spec(1), _row_spec(din)],
        out_specs=_row_spec(din), out_shape=_shape(din))(s2c, s2c, da, db, r)


def _tc_ha(s2c, da, db, r, w):
    din = s2c.shape[2]
    dout = w.shape[0]
    return pl.pallas_call(
        _ha_body, grid=(N // BM,),
        in_specs=[_part_spec(din, 0), _part_spec(din, 1), _row_spec(1),
                  _row_spec(1), _row_spec(din), _full_spec(dout, din)],
        out_specs=[_row_spec(din), _row_spec(dout)],
        out_shape=[_shape(din), _shape(dout)])(s2c, s2c, da, db, r, w)


def _tc_fin(s2c, da, db, r, w):
    din = s2c.shape[2]
    dout = w.shape[0]
    return pl.pallas_call(
        _fin_body, grid=(N // BM,),
        in_specs=[_part_spec(din, 0), _part_spec(din, 1), _row_spec(1),
                  _row_spec(1), _row_spec(dout), _full_spec(dout, din)],
        out_specs=_row_spec(dout), out_shape=_shape(dout))(
            s2c, s2c, da, db, r, w)


@jax.jit
def kernel(x, edge_index, Wl0, bl0, Wr0, Wl1, bl1, Wr1, Wl2, bl2, Wr2):
    edges = jnp.pad(edge_index, ((0, 0), (0, EPAD - E)))
    edges = edges.reshape(2, NW, NCH, CHUNK)

    zrows128 = jnp.zeros((CHUNK, D_HID), jnp.float32)
    zdeg = jnp.zeros((RPT,), jnp.float32)
    ones_in = jnp.ones((CHUNK,), jnp.float32)

    sc0 = _make_sc_scatter(D_HID, True)
    sc1 = _make_sc_scatter(D_HID, False)

    a0 = _tc_mm(x, Wl0)
    s0, deg = sc0(a0, edges, zrows128, zdeg, ones_in)
    r0 = _tc_mmb(x, Wr0, bl0)
    da = deg[0, :N, None]
    db = deg[1, :N, None]

    h1, a1 = _tc_ha(s0, da, db, r0, Wl1)
    (s1,) = sc1(a1, edges, zrows128)
    r1 = _tc_mmb(h1, Wr1, bl1)

    h2 = _tc_h(s1, da, db, r1)
    (s2,) = sc1(h2, edges, zrows128)
    r2 = _tc_mmb(h2, Wr2, bl2)

    return _tc_fin(s2, da, db, r2, Wl2)


# speedup vs baseline: 1.1026x; 1.0022x over previous
import jax
import jax.numpy as jnp
from jax import lax
from jax.experimental import pallas as pl
from jax.experimental.pallas import tpu as pltpu
from jax.experimental.pallas import tpu_sc as plsc

N = 10000
E = 160000
D_IN = 256
D_HID = 128
D_OUT = 64

NC = 2
NS = 16
NW = NC * NS
CHUNK = 128
NCH = (E + NW * CHUNK - 1) // (NW * CHUNK)
EPAD = NW * NCH * CHUNK
NP = 10240
RPT = NP // NS


def _sc_body(with_deg, d, table, edges, zrows, zdeg, ones_in,
             out_acc, out_deg, src_v, dst_v, rows, ones_v, acc_sh,
             deg_sh, gsems, ssems, esems):
    c = lax.axis_index("c")
    s = lax.axis_index("s")
    wid = s * NC + c
    tail = (E - (NW - 1) * NCH * CHUNK) // CHUNK
    trip2 = jnp.where(wid == NW - 1, tail // 2, NCH // 2)

    pltpu.sync_copy(zrows, rows[0])
    pltpu.sync_copy(edges.at[0, wid], src_v)
    pltpu.sync_copy(edges.at[1, wid], dst_v)
    for z in range(RPT // CHUNK):
        pltpu.sync_copy(rows[0], acc_sh.at[pl.ds(s * RPT + z * CHUNK, CHUNK)])
    if with_deg:
        pltpu.sync_copy(zdeg, deg_sh.at[pl.ds(s * RPT, RPT)])
        pltpu.sync_copy(ones_in, ones_v)
    plsc.subcore_barrier()


    def block(j, width, carry):
        gh = []
        for k in range(width):
            @pl.when(j >= 2)
            def _():
                pltpu.make_async_copy(table.at[src_v.at[j + k]], rows[k],
                                      ssems[k]).wait()
                if with_deg:
                    pltpu.make_async_copy(ones_in, ones_v, esems[k]).wait()
            gh.append(pltpu.async_copy(table.at[src_v.at[j + k]], rows[k],
                                       gsems[k]))
        for k in range(width):
            gh[k].wait()
            pltpu.async_copy(rows[k], acc_sh.at[dst_v.at[j + k]],
                             ssems[k], add=True)
            if with_deg:
                pltpu.async_copy(ones_v, deg_sh.at[dst_v.at[j + k]],
                                 esems[k], add=True)
        return carry

    lax.fori_loop(0, trip2, lambda j2, cy: block(2 * j2, 2, cy), 0)
    for k in range(2):
        pltpu.make_async_copy(table.at[src_v.at[0]], rows[k],
                              ssems[k]).wait()
        if with_deg:
            pltpu.make_async_copy(ones_in, ones_v, esems[k]).wait()
    plsc.subcore_barrier()

    pltpu.sync_copy(acc_sh.at[pl.ds(s * RPT, RPT)],
                    out_acc.at[c, pl.ds(s * RPT, RPT)])
    if with_deg:
        pltpu.sync_copy(deg_sh.at[pl.ds(s * RPT, RPT)],
                        out_deg.at[c, pl.ds(s * RPT, RPT)])


def _make_sc_scatter(d, with_deg):
    mesh = plsc.VectorSubcoreMesh(core_axis_name="c", subcore_axis_name="s",
                                  num_cores=NC, num_subcores=NS)
    out_type = [jax.ShapeDtypeStruct((NC, NP, d), jnp.float32)]
    if with_deg:
        out_type.append(jax.ShapeDtypeStruct((NC, NP), jnp.float32))
    scratch = [
        pltpu.VMEM((NCH, CHUNK), jnp.int32),
        pltpu.VMEM((NCH, CHUNK), jnp.int32),
        pltpu.VMEM((CHUNK, d), jnp.float32),
        pltpu.VMEM((CHUNK, d), jnp.float32),
        pltpu.VMEM((CHUNK,), jnp.float32),
        pltpu.VMEM_SHARED((NP, d), jnp.float32),
        pltpu.VMEM_SHARED((NP,), jnp.float32),
    ] + [pltpu.SemaphoreType.DMA] * 6

    if with_deg:
        def body(table, edges, zrows, zdeg, ones_in, out_acc, out_deg,
                 src_v, dst_v, r0, r1, ones_v, acc_sh, deg_sh, *sems):
            _sc_body(True, d, table, edges, zrows, zdeg, ones_in,
                     out_acc, out_deg, src_v, dst_v, (r0, r1), ones_v,
                     acc_sh, deg_sh, sems[0:2], sems[2:4], sems[4:6])
    else:
        def body(table, edges, zrows, out_acc,
                 src_v, dst_v, r0, r1, ones_v, acc_sh, deg_sh, *sems):
            _sc_body(False, d, table, edges, zrows, None, None,
                     out_acc, None, src_v, dst_v, (r0, r1), ones_v,
                     acc_sh, deg_sh, sems[0:2], sems[2:4], sems[4:6])

    return pl.kernel(body, out_type=out_type, mesh=mesh,
                     scratch_types=scratch)


BM = 2000


def _dotT(x, w):
    return lax.dot_general(x, w, (((1,), (1,)), ((), ())),
                           preferred_element_type=jnp.float32)


def _mm_body(x_ref, w_ref, o_ref):
    o_ref[...] = _dotT(x_ref[...], w_ref[...])


def _mmb_body(x_ref, w_ref, b_ref, o_ref):
    o_ref[...] = _dotT(x_ref[...], w_ref[...]) + b_ref[...]


def _h_body(sa_ref, sb_ref, da_ref, db_ref, r_ref, h_ref):
    invd = 1.0 / jnp.maximum(da_ref[...] + db_ref[...], 1.0)
    h_ref[...] = jnp.maximum((sa_ref[0] + sb_ref[0]) * invd + r_ref[...], 0.0)


def _ha_body(sa_ref, sb_ref, da_ref, db_ref, r_ref, w_ref, h_ref, a_ref):
    invd = 1.0 / jnp.maximum(da_ref[...] + db_ref[...], 1.0)
    h = jnp.maximum((sa_ref[0] + sb_ref[0]) * invd + r_ref[...], 0.0)
    h_ref[...] = h
    a_ref[...] = _dotT(h, w_ref[...])


def _fin_body(sa_ref, sb_ref, da_ref, db_ref, r_ref, w_ref, o_ref):
    invd = 1.0 / jnp.maximum(da_ref[...] + db_ref[...], 1.0)
    mean = (sa_ref[0] + sb_ref[0]) * invd
    o_ref[...] = _dotT(mean, w_ref[...]) + r_ref[...]


def _row_spec(dcol):
    return pl.BlockSpec((BM, dcol), lambda i: (i, 0))


def _part_spec(dcol, core):
    return pl.BlockSpec((1, BM, dcol), lambda i, c=core: (c, i, 0))


def _full_spec(r, c):
    return pl.BlockSpec((r, c), lambda i: (0, 0))


def _shape(dcol):
    return jax.ShapeDtypeStruct((N, dcol), jnp.float32)


def _tc_mm(x, w):
    dout, din = w.shape
    return pl.pallas_call(
        _mm_body, grid=(N // BM,),
        in_specs=[_row_spec(din), _full_spec(dout, din)],
        out_specs=_row_spec(dout), out_shape=_shape(dout))(x, w)


def _tc_mmb(x, w, b):
    dout, din = w.shape
    return pl.pallas_call(
        _mmb_body, grid=(N // BM,),
        in_specs=[_row_spec(din), _full_spec(dout, din), _full_spec(1, dout)],
        out_specs=_row_spec(dout), out_shape=_shape(dout))(x, w, b[None, :])


def _tc_h(s2c, da, db, r):
    din = s2c.shape[2]
    return pl.pallas_call(
        _h_body, grid=(N // BM,),
        in_specs=[_part_spec(din, 0), _part_spec(din, 1), _row_spec(1),
                  _row_spec(1), _row_spec(din)],
        out_specs=_row_spec(din), out_shape=_shape(din))(s2c, s2c, da, db, r)


def _tc_ha(s2c, da, db, r, w):
    din = s2c.shape[2]
    dout = w.shape[0]
    return pl.pallas_call(
        _ha_body, grid=(N // BM,),
        in_specs=[_part_spec(din, 0), _part_spec(din, 1), _row_spec(1),
                  _row_spec(1), _row_spec(din), _full_spec(dout, din)],
        out_specs=[_row_spec(din), _row_spec(dout)],
        out_shape=[_shape(din), _shape(dout)])(s2c, s2c, da, db, r, w)


def _tc_fin(s2c, da, db, r, w):
    din = s2c.shape[2]
    dout = w.shape[0]
    return pl.pallas_call(
        _fin_body, grid=(N // BM,),
        in_specs=[_part_spec(din, 0), _part_spec(din, 1), _row_spec(1),
                  _row_spec(1), _row_spec(dout), _full_spec(dout, din)],
        out_specs=_row_spec(dout), out_shape=_shape(dout))(
            s2c, s2c, da, db, r, w)


@jax.jit
def kernel(x, edge_index, Wl0, bl0, Wr0, Wl1, bl1, Wr1, Wl2, bl2, Wr2):
    edges = jnp.pad(edge_index, ((0, 0), (0, EPAD - E)))
    edges = edges.reshape(2, NW, NCH, CHUNK)

    zrows128 = jnp.zeros((CHUNK, D_HID), jnp.float32)
    zdeg = jnp.zeros((RPT,), jnp.float32)
    ones_in = jnp.ones((CHUNK,), jnp.float32)

    sc0 = _make_sc_scatter(D_HID, True)
    sc1 = _make_sc_scatter(D_HID, False)

    a0 = _tc_mm(x, Wl0)
    s0, deg = sc0(a0, edges, zrows128, zdeg, ones_in)
    r0 = _tc_mmb(x, Wr0, bl0)
    da = deg[0, :N, None]
    db = deg[1, :N, None]

    h1, a1 = _tc_ha(s0, da, db, r0, Wl1)
    (s1,) = sc1(a1, edges, zrows128)
    r1 = _tc_mmb(h1, Wr1, bl1)

    h2 = _tc_h(s1, da, db, r1)
    (s2,) = sc1(h2, edges, zrows128)
    r2 = _tc_mmb(h2, Wr2, bl2)

    return _tc_fin(s2, da, db, r2, Wl2)
